# baseline (speedup 1.0000x reference)
import functools
import math

import jax
import jax.numpy as jnp
from jax.experimental import pallas as pl
from jax.experimental.pallas import tpu as pltpu
from jax.experimental.pallas import tpu_sc as plsc

N = 10000
D = 128
K = 27
BN = 512
NPAD = 10240
GW = 128


def _prep_body(emb27_ref, sh27_ref, weight_ref, wsc_ref, o_ref):
    w = jnp.dot(emb27_ref[...], weight_ref[...],
                preferred_element_type=jnp.float32)
    scale = 1.0 / (K * math.sqrt(float(D)))
    o_ref[...] = w * sh27_ref[...] * scale
    o_ref[13:14, :] = o_ref[13:14, :] + wsc_ref[...] * (1.0 / math.sqrt(float(D)))


def _prep(emb27, sh27, weight, wsc_row):
    return pl.pallas_call(
        _prep_body,
        out_shape=jax.ShapeDtypeStruct((K, D * D), jnp.float32),
    )(emb27, sh27, weight, wsc_row)


def _sc_gather(x_pad, idx_cols):
    mesh = plsc.VectorSubcoreMesh(core_axis_name="c", subcore_axis_name="s")
    out_type = jax.ShapeDtypeStruct((NPAD, K * D), x_pad.dtype)

    @functools.partial(pl.kernel, out_type=out_type, mesh=mesh)
    def gather_kernel(x_hbm, i_hbm, o_hbm):
        def body(i_vmem, o_vmem):
            pltpu.sync_copy(x_hbm.at[i_vmem.at[0]], o_vmem)

        pltpu.emit_pipeline(
            body,
            grid=(K, NPAD // GW),
            in_specs=[pl.BlockSpec((1, GW), index_map=lambda k, i: (k, i))],
            out_specs=[pl.BlockSpec((GW, D), index_map=lambda k, i: (i, k))],
            core_axis_name=("c", "s"),
            dimension_semantics=(pltpu.PARALLEL, pltpu.PARALLEL),
        )(i_hbm, o_hbm)

    return gather_kernel(x_pad, idx_cols)


def _conv_body(g_ref, a_ref, o_ref):
    o_ref[...] = jnp.dot(g_ref[...], a_ref[...],
                         preferred_element_type=jnp.float32)


def _conv(G, Astack):
    return pl.pallas_call(
        _conv_body,
        grid=(NPAD // BN,),
        in_specs=[
            pl.BlockSpec((BN, K * D), lambda i: (i, 0)),
            pl.BlockSpec((K * D, D), lambda i: (0, 0)),
        ],
        out_specs=pl.BlockSpec((BN, D), lambda i: (i, 0)),
        out_shape=jax.ShapeDtypeStruct((NPAD, D), jnp.float32),
    )(G, Astack)


def kernel(x, W_sc, weight, emb, sh, neigh_idx):
    x = x.astype(jnp.float32)
    x_pad = jnp.concatenate([x, jnp.zeros((1, D), x.dtype)], axis=0)
    idx = neigh_idx.astype(jnp.int32)
    idx_cols = jnp.pad(idx.T, ((0, 0), (0, NPAD - N)), constant_values=N)
    emb27 = emb.transpose(2, 1, 0, 3).reshape(K, -1)
    sh27 = sh[..., 0].transpose(2, 1, 0).reshape(K, 1)
    wsc_row = W_sc.reshape(1, D * D)

    Astack = _prep(emb27, sh27, weight, wsc_row).reshape(K * D, D)
    G = _sc_gather(x_pad, idx_cols)
    out = _conv(G, Astack)
    return out[:N]
